# baseline (speedup 1.0000x reference)
import functools

import jax
import jax.numpy as jnp
from jax import lax
from jax.experimental import pallas as pl
from jax.experimental.pallas import tpu as pltpu
from jax.experimental.pallas import tpu_sc as plsc

N = 10000
E = 320000
IN = 128
H = 128
M = 64

NC = 2
NS = 16
NW = NC * NS
CH = 128
NCH_A = 78
NCH_LAST = 82
EPW = NCH_A * CH
RPT = 624
RLAST = N - (NS - 1) * RPT

BN = 1000
BE = 8000


def _mesh():
    return plsc.VectorSubcoreMesh(
        core_axis_name="c", subcore_axis_name="s",
        num_cores=NC, num_subcores=NS)


def _zero_own_rows(sid, z_h, acc):
    r0 = pl.multiple_of(sid * RPT, 8)

    @pl.when(sid == NS - 1)
    def _():
        pltpu.sync_copy(z_h, acc.at[pl.ds(r0, RLAST)])

    @pl.when(sid < NS - 1)
    def _():
        pltpu.sync_copy(z_h.at[pl.ds(0, RPT)], acc.at[pl.ds(r0, RPT)])


def _write_own_rows(sid, cid, acc, out_h):
    r0 = pl.multiple_of(sid * RPT, 8)

    @pl.when(sid == NS - 1)
    def _():
        pltpu.sync_copy(acc.at[pl.ds(r0, RLAST)], out_h.at[cid, pl.ds(r0, RLAST)])

    @pl.when(sid < NS - 1)
    def _():
        pltpu.sync_copy(acc.at[pl.ds(r0, RPT)], out_h.at[cid, pl.ds(r0, RPT)])


def _sc_scatter(u, src, dst, zrows):
    width = u.shape[1]

    @functools.partial(
        pl.kernel,
        out_type=jax.ShapeDtypeStruct((NC, N, width), jnp.float32),
        mesh=_mesh(),
        scratch_types=[
            pltpu.VMEM((2, CH), jnp.int32),
            pltpu.VMEM((2, CH), jnp.int32),
            pltpu.VMEM((2, CH, width), jnp.float32),
            pltpu.VMEM_SHARED((N, width), jnp.float32),
            pltpu.SemaphoreType.DMA((2,)),
            pltpu.SemaphoreType.DMA((2,)),
        ])
    def k(u_h, src_h, dst_h, z_h, out_h, sidx, didx, rows, acc, gsem, isem):
        cid = lax.axis_index("c")
        sid = lax.axis_index("s")
        wid = cid * NS + sid
        _zero_own_rows(sid, z_h, acc)
        plsc.subcore_barrier()
        ebase = wid * EPW
        ncu = jnp.where(wid == NW - 1, NCH_LAST, NCH_A)

        pltpu.sync_copy(src_h.at[pl.ds(ebase, CH)], sidx.at[0])
        pltpu.sync_copy(dst_h.at[pl.ds(ebase, CH)], didx.at[0])
        b1_ = pl.multiple_of(ebase + CH, 8)
        pltpu.async_copy(src_h.at[pl.ds(b1_, CH)], sidx.at[1], isem.at[1])
        pltpu.async_copy(dst_h.at[pl.ds(b1_, CH)], didx.at[1], isem.at[1])
        pltpu.async_copy(u_h.at[sidx.at[0]], rows.at[0], gsem.at[0])

        @pl.loop(0, ncu)
        def _(j):
            p = lax.rem(j, 2)
            q = 1 - p

            @pl.when(j < ncu - 1)
            def _():
                pltpu.make_async_copy(
                    src_h.at[pl.ds(0, CH)], sidx.at[q], isem.at[q]).wait()
                pltpu.make_async_copy(
                    dst_h.at[pl.ds(0, CH)], didx.at[q], isem.at[q]).wait()
                pltpu.async_copy(u_h.at[sidx.at[q]], rows.at[q], gsem.at[q])

            pltpu.make_async_copy(
                u_h.at[pl.ds(0, CH)], rows.at[p], gsem.at[p]).wait()
            pltpu.sync_copy(rows.at[p], acc.at[didx.at[p]], add=True)

            @pl.when(j < ncu - 2)
            def _():
                nb = pl.multiple_of(ebase + (j + 2) * CH, 8)
                pltpu.async_copy(src_h.at[pl.ds(nb, CH)], sidx.at[p], isem.at[p])
                pltpu.async_copy(dst_h.at[pl.ds(nb, CH)], didx.at[p], isem.at[p])

        plsc.subcore_barrier()
        _write_own_rows(sid, cid, acc, out_h)

    return k(u, src, dst, zrows)


def _sc_degree(dst, zrows, ones_h):
    @functools.partial(
        pl.kernel,
        out_type=jax.ShapeDtypeStruct((NC, N, H), jnp.float32),
        mesh=_mesh(),
        scratch_types=[
            pltpu.VMEM((2, CH), jnp.int32),
            pltpu.VMEM((CH, H), jnp.float32),
            pltpu.VMEM_SHARED((N, H), jnp.float32),
            pltpu.SemaphoreType.DMA((2,)),
        ])
    def k(dst_h, z_h, ones_hbm, out_h, didx, onesb, acc, isem):
        cid = lax.axis_index("c")
        sid = lax.axis_index("s")
        wid = cid * NS + sid
        _zero_own_rows(sid, z_h, acc)
        pltpu.sync_copy(ones_hbm, onesb)
        plsc.subcore_barrier()
        ebase = wid * EPW
        ncu = jnp.where(wid == NW - 1, NCH_LAST, NCH_A)

        pltpu.sync_copy(dst_h.at[pl.ds(ebase, CH)], didx.at[0])

        @pl.loop(0, ncu)
        def _(j):
            p = lax.rem(j, 2)
            q = 1 - p

            @pl.when(j < ncu - 1)
            def _():
                nb = pl.multiple_of(ebase + (j + 1) * CH, 8)
                pltpu.async_copy(dst_h.at[pl.ds(nb, CH)], didx.at[q], isem.at[q])

            pltpu.sync_copy(onesb, acc.at[didx.at[p]], add=True)

            @pl.when(j < ncu - 1)
            def _():
                pltpu.make_async_copy(
                    dst_h.at[pl.ds(0, CH)], didx.at[q], isem.at[q]).wait()

        plsc.subcore_barrier()
        _write_own_rows(sid, cid, acc, out_h)

    return k(dst, zrows, ones_h)


def _sc_route_gather(ab, src, dst):
    CHR = 64
    NCR_A = EPW // CHR
    NCR_LAST = NCR_A + 8

    @functools.partial(
        pl.kernel,
        out_type=jax.ShapeDtypeStruct((E, M), jnp.float32),
        mesh=_mesh(),
        scratch_types=[
            pltpu.VMEM((2, CHR), jnp.int32),
            pltpu.VMEM((2, CHR), jnp.int32),
            pltpu.VMEM((2, CHR, H), jnp.float32),
            pltpu.VMEM((2, CHR, H), jnp.float32),
            pltpu.VMEM((2, CHR, M), jnp.float32),
            pltpu.VMEM_SHARED((N, H), jnp.float32),
            pltpu.SemaphoreType.DMA((2,)),
            pltpu.SemaphoreType.DMA((2,)),
            pltpu.SemaphoreType.DMA((2,)),
            pltpu.SemaphoreType.DMA((2,)),
        ])
    def k(ab_h, src_h, dst_h, g_h, sidx, didx, ra, rb, go, abs_, gsa, gsb, isem, wsem):
        cid = lax.axis_index("c")
        sid = lax.axis_index("s")
        wid = cid * NS + sid
        ebase = wid * EPW
        ncu = jnp.where(wid == NW - 1, NCR_LAST, NCR_A)

        r0 = pl.multiple_of(sid * RPT, 8)

        @pl.when(sid == NS - 1)
        def _():
            pltpu.sync_copy(ab_h.at[pl.ds(r0, RLAST)], abs_.at[pl.ds(r0, RLAST)])

        @pl.when(sid < NS - 1)
        def _():
            pltpu.sync_copy(ab_h.at[pl.ds(r0, RPT)], abs_.at[pl.ds(r0, RPT)])

        plsc.subcore_barrier()

        pltpu.sync_copy(src_h.at[pl.ds(ebase, CHR)], sidx.at[0])
        pltpu.sync_copy(dst_h.at[pl.ds(ebase, CHR)], didx.at[0])
        b1_ = pl.multiple_of(ebase + CHR, 8)
        pltpu.async_copy(src_h.at[pl.ds(b1_, CHR)], sidx.at[1], isem.at[1])
        pltpu.async_copy(dst_h.at[pl.ds(b1_, CHR)], didx.at[1], isem.at[1])
        pltpu.async_copy(abs_.at[sidx.at[0]], ra.at[0], gsa.at[0])
        pltpu.async_copy(abs_.at[didx.at[0]], rb.at[0], gsb.at[0])

        @pl.loop(0, ncu)
        def _(j):
            p = lax.rem(j, 2)
            q = 1 - p
            base = pl.multiple_of(ebase + j * CHR, 8)

            @pl.when(j < ncu - 1)
            def _():
                pltpu.make_async_copy(
                    src_h.at[pl.ds(0, CHR)], sidx.at[q], isem.at[q]).wait()
                pltpu.make_async_copy(
                    dst_h.at[pl.ds(0, CHR)], didx.at[q], isem.at[q]).wait()
                pltpu.async_copy(abs_.at[sidx.at[q]], ra.at[q], gsa.at[q])
                pltpu.async_copy(abs_.at[didx.at[q]], rb.at[q], gsb.at[q])

            pltpu.make_async_copy(
                abs_.at[pl.ds(0, CHR)], ra.at[p], gsa.at[p]).wait()
            pltpu.make_async_copy(
                abs_.at[pl.ds(0, CHR)], rb.at[p], gsb.at[p]).wait()

            @pl.when(j >= 2)
            def _():
                pltpu.make_async_copy(
                    go.at[p], g_h.at[pl.ds(0, CHR)], wsem.at[p]).wait()

            @pl.loop(0, CHR, unroll=4)
            def _(r):
                for c in range(M // 16):
                    s = c * 16
                    go[p, r, pl.ds(s, 16)] = (
                        ra[p, r, pl.ds(s, 16)] + rb[p, r, pl.ds(M + s, 16)])

            pltpu.async_copy(go.at[p], g_h.at[pl.ds(base, CHR)], wsem.at[p])

            @pl.when(j < ncu - 2)
            def _():
                nb = pl.multiple_of(ebase + (j + 2) * CHR, 8)
                pltpu.async_copy(src_h.at[pl.ds(nb, CHR)], sidx.at[p], isem.at[p])
                pltpu.async_copy(dst_h.at[pl.ds(nb, CHR)], didx.at[p], isem.at[p])

        pltpu.make_async_copy(
            go.at[0], g_h.at[pl.ds(0, CHR)], wsem.at[0]).wait()
        pltpu.make_async_copy(
            go.at[1], g_h.at[pl.ds(0, CHR)], wsem.at[1]).wait()

    return k(ab, src, dst)


def _rep(shape):
    return pl.BlockSpec(shape, lambda i: tuple(0 for _ in shape))


def _tc_head(x, wlT, wdT, bias):
    def body(x_ref, wl_ref, wd_ref, b_ref, u_ref, d_ref):
        xb = x_ref[...]
        u_ref[...] = jnp.dot(xb, wl_ref[...], preferred_element_type=jnp.float32)
        d_ref[...] = jnp.dot(xb, wd_ref[...], preferred_element_type=jnp.float32) + b_ref[...]

    return pl.pallas_call(
        body,
        grid=(N // BN,),
        in_specs=[
            pl.BlockSpec((BN, IN), lambda i: (i, 0)),
            _rep((IN, H)), _rep((IN, H)), _rep((1, H)),
        ],
        out_specs=(pl.BlockSpec((BN, H), lambda i: (i, 0)),
                   pl.BlockSpec((BN, H), lambda i: (i, 0))),
        out_shape=(jax.ShapeDtypeStruct((N, H), jnp.float32),
                   jax.ShapeDtypeStruct((N, H), jnp.float32)),
    )(x, wlT, wdT, bias)


def _layer_out(sa, sb, dega, degb, d, g, be):
    deg = jnp.maximum(dega[:, :1] + degb[:, :1], 1.0)
    pre = (sa + sb) / deg + d
    mu = jnp.mean(pre, axis=-1, keepdims=True)
    var = jnp.mean((pre - mu) ** 2, axis=-1, keepdims=True)
    return jnp.maximum((pre - mu) * lax.rsqrt(var + 1e-5) * g + be, 0.0)


def _tc_mid(sa, sb, dega, degb, d, g, be, wlT, wdT, bias):
    win = d.shape[1]
    wu = wlT.shape[1]
    wd = wdT.shape[1]

    def body(sa_ref, sb_ref, da_ref, db_ref, d_ref, g_ref, be_ref,
             wl_ref, wd_ref, b_ref, u_ref, dn_ref):
        h = _layer_out(sa_ref[...], sb_ref[...], da_ref[...], db_ref[...],
                       d_ref[...], g_ref[...], be_ref[...])
        u_ref[...] = jnp.dot(h, wl_ref[...], preferred_element_type=jnp.float32)
        dn_ref[...] = jnp.dot(h, wd_ref[...], preferred_element_type=jnp.float32) + b_ref[...]

    return pl.pallas_call(
        body,
        grid=(N // BN,),
        in_specs=[
            pl.BlockSpec((BN, win), lambda i: (i, 0)),
            pl.BlockSpec((BN, win), lambda i: (i, 0)),
            pl.BlockSpec((BN, H), lambda i: (i, 0)),
            pl.BlockSpec((BN, H), lambda i: (i, 0)),
            pl.BlockSpec((BN, win), lambda i: (i, 0)),
            _rep((1, win)), _rep((1, win)),
            _rep((win, wu)), _rep((win, wd)), _rep((1, wd)),
        ],
        out_specs=(pl.BlockSpec((BN, wu), lambda i: (i, 0)),
                   pl.BlockSpec((BN, wd), lambda i: (i, 0))),
        out_shape=(jax.ShapeDtypeStruct((N, wu), jnp.float32),
                   jax.ShapeDtypeStruct((N, wd), jnp.float32)),
    )(sa, sb, dega, degb, d, g, be, wlT, wdT, bias)


def _tc_final(sa, sb, dega, degb, d3, hp, g3, be3,
              wihT, bih, whhT, bhh, vw1T, vb1, vw2, vb2,
              aw1T, ab1, aw2, ab2, rw1abT, rb1ab):
    grid = N // BN

    def body(sa_ref, sb_ref, da_ref, db_ref, d_ref, hp_ref, g_ref, be_ref,
             wih_ref, bih_ref, whh_ref, bhh_ref, vw1_ref, vb1_ref, vw2_ref, vb2_ref,
             aw1_ref, ab1_ref, aw2_ref, ab2_ref, rw1ab_ref, rb1ab_ref,
             ht_ref, vuln_ref, ab_ref, adapt_ref, acc_ref):
        i = pl.program_id(0)
        e = _layer_out(sa_ref[...][:, :M], sb_ref[...][:, :M],
                       da_ref[...], db_ref[...],
                       d_ref[...], g_ref[...], be_ref[...])
        hp_b = hp_ref[...]
        gi = jnp.dot(e, wih_ref[...], preferred_element_type=jnp.float32) + bih_ref[...]
        gh = jnp.dot(hp_b, whh_ref[...], preferred_element_type=jnp.float32) + bhh_ref[...]
        r = jax.nn.sigmoid(gi[:, :M] + gh[:, :M])
        z = jax.nn.sigmoid(gi[:, M:2 * M] + gh[:, M:2 * M])
        ng = jnp.tanh(gi[:, 2 * M:] + r * gh[:, 2 * M:])
        ht = (1.0 - z) * ng + z * hp_b
        ht_ref[...] = ht
        hv = jnp.maximum(jnp.dot(ht, vw1_ref[...], preferred_element_type=jnp.float32)
                         + vb1_ref[...], 0.0)
        vuln_ref[...] = jax.nn.sigmoid(
            jnp.sum(hv * vw2_ref[...], axis=-1, keepdims=True) + vb2_ref[...])
        ab_ref[...] = (jnp.dot(ht, rw1ab_ref[...], preferred_element_type=jnp.float32)
                       + rb1ab_ref[...])

        @pl.when(i == 0)
        def _():
            acc_ref[...] = jnp.zeros_like(acc_ref)

        acc_ref[...] += jnp.sum(ht, axis=0, keepdims=True)

        @pl.when(i == grid - 1)
        def _():
            pooled = acc_ref[...] / N
            t = jnp.maximum(jnp.dot(pooled, aw1_ref[...], preferred_element_type=jnp.float32)
                            + ab1_ref[...], 0.0)
            adapt_ref[...] = jax.nn.sigmoid(
                jnp.sum(t * aw2_ref[...], axis=-1, keepdims=True) + ab2_ref[...])

    return pl.pallas_call(
        body,
        grid=(grid,),
        in_specs=[
            pl.BlockSpec((BN, H), lambda i: (i, 0)),
            pl.BlockSpec((BN, H), lambda i: (i, 0)),
            pl.BlockSpec((BN, H), lambda i: (i, 0)),
            pl.BlockSpec((BN, H), lambda i: (i, 0)),
            pl.BlockSpec((BN, M), lambda i: (i, 0)),
            pl.BlockSpec((BN, M), lambda i: (i, 0)),
            _rep((1, M)), _rep((1, M)),
            _rep((M, 3 * M)), _rep((1, 3 * M)), _rep((M, 3 * M)), _rep((1, 3 * M)),
            _rep((M, M)), _rep((1, M)), _rep((1, M)), _rep((1, 1)),
            _rep((M, M)), _rep((1, M)), _rep((1, M)), _rep((1, 1)),
            _rep((M, H)), _rep((1, H)),
        ],
        out_specs=(pl.BlockSpec((BN, M), lambda i: (i, 0)),
                   pl.BlockSpec((BN, 1), lambda i: (i, 0)),
                   pl.BlockSpec((BN, H), lambda i: (i, 0)),
                   pl.BlockSpec((1, 1), lambda i: (0, 0))),
        out_shape=(jax.ShapeDtypeStruct((N, M), jnp.float32),
                   jax.ShapeDtypeStruct((N, 1), jnp.float32),
                   jax.ShapeDtypeStruct((N, H), jnp.float32),
                   jax.ShapeDtypeStruct((1, 1), jnp.float32)),
        scratch_shapes=[pltpu.VMEM((1, M), jnp.float32)],
    )(sa, sb, dega, degb, d3, hp, g3, be3, wihT, bih, whhT, bhh,
      vw1T, vb1, vw2, vb2, aw1T, ab1, aw2, ab2, rw1abT, rb1ab)


def _tc_route(g, ea, we, rw2, rb2):
    def body(g_ref, ea_ref, we_ref, rw2_ref, rb2_ref, out_ref):
        pre = jnp.maximum(g_ref[...] + ea_ref[...] * we_ref[...], 0.0)
        out_ref[...] = jax.nn.sigmoid(
            jnp.sum(pre * rw2_ref[...], axis=-1, keepdims=True) + rb2_ref[...])

    return pl.pallas_call(
        body,
        grid=(E // BE,),
        in_specs=[
            pl.BlockSpec((BE, M), lambda i: (i, 0)),
            pl.BlockSpec((BE, 1), lambda i: (i, 0)),
            _rep((1, M)), _rep((1, M)), _rep((1, 1)),
        ],
        out_specs=pl.BlockSpec((BE, 1), lambda i: (i, 0)),
        out_shape=jax.ShapeDtypeStruct((E, 1), jnp.float32),
    )(g, ea, we, rw2, rb2)


def kernel(x, edge_index, edge_attr, h_prev, params):
    p = params
    f32 = jnp.float32
    src = edge_index[0]
    dst = edge_index[1]

    z128 = jnp.zeros((RLAST, H), f32)
    ones128 = jnp.ones((CH, H), f32)

    w1lT = p['W1l'].T
    wd1T = (p['W1r'] + p['S1']).T
    b1 = p['b1l'][None, :]
    w2lT = p['W2l'].T
    wd2T = (p['W2r'] + p['S2']).T
    b2 = p['b2l'][None, :]
    w3lT_pad = jnp.concatenate(
        [p['W3l'].T, jnp.zeros((H, H - M), f32)], axis=1)
    wd3T = (p['W3r'] + p['S3']).T
    b3 = p['b3l'][None, :]
    g1 = p['g1'][None, :]; be1 = p['be1'][None, :]
    g2 = p['g2'][None, :]; be2 = p['be2'][None, :]
    g3 = p['g3'][None, :]; be3 = p['be3'][None, :]
    wihT = p['Wih'].T; bih = p['bih'][None, :]
    whhT = p['Whh'].T; bhh = p['bhh'][None, :]
    vw1T = p['Vw1'].T; vb1 = p['Vb1'][None, :]
    vw2 = p['Vw2']; vb2 = p['Vb2'][None, :]
    aw1T = p['Aw1'].T; ab1 = p['Ab1'][None, :]
    aw2 = p['Aw2']; ab2 = p['Ab2'][None, :]
    rw1abT = jnp.concatenate(
        [p['Rw1'][:, :M].T, p['Rw1'][:, M:2 * M].T], axis=1)
    rb1ab = jnp.concatenate([p['Rb1'], jnp.zeros((M,), f32)])[None, :]
    we = p['Rw1'][:, 2 * M:].T
    rw2 = p['Rw2']; rb2 = p['Rb2'][None, :]

    degp = _sc_degree(dst, z128, ones128)
    dega, degb = degp[0], degp[1]
    u1, d1 = _tc_head(x, w1lT, wd1T, b1)
    s1 = _sc_scatter(u1, src, dst, z128)
    u2, d2 = _tc_mid(s1[0], s1[1], dega, degb, d1, g1, be1, w2lT, wd2T, b2)
    s2 = _sc_scatter(u2, src, dst, z128)
    u3, d3 = _tc_mid(s2[0], s2[1], dega, degb, d2, g2, be2, w3lT_pad, wd3T, b3)
    s3 = _sc_scatter(u3, src, dst, z128)
    ht, vuln, ab, adapt = _tc_final(
        s3[0], s3[1], dega, degb, d3, h_prev, g3, be3,
        wihT, bih, whhT, bhh, vw1T, vb1, vw2, vb2,
        aw1T, ab1, aw2, ab2, rw1abT, rb1ab)
    g = _sc_route_gather(ab, src, dst)
    route = _tc_route(g, edge_attr, we, rw2, rb2)
    return vuln[:, 0], adapt[:, 0], route[:, 0], ht


# speedup vs baseline: 1.0351x; 1.0351x over previous
import functools

import jax
import jax.numpy as jnp
from jax import lax
from jax.experimental import pallas as pl
from jax.experimental.pallas import tpu as pltpu
from jax.experimental.pallas import tpu_sc as plsc

N = 10000
E = 320000
IN = 128
H = 128
M = 64

NC = 2
NS = 16
NW = NC * NS
CH = 128
NCH_A = 78
NCH_LAST = 82
EPW = NCH_A * CH
RPT = 624
RLAST = N - (NS - 1) * RPT

BN = 2000
BE = 16000


def _mesh():
    return plsc.VectorSubcoreMesh(
        core_axis_name="c", subcore_axis_name="s",
        num_cores=NC, num_subcores=NS)


def _zero_own_rows(sid, z_h, acc):
    r0 = pl.multiple_of(sid * RPT, 8)

    @pl.when(sid == NS - 1)
    def _():
        pltpu.sync_copy(z_h, acc.at[pl.ds(r0, RLAST)])

    @pl.when(sid < NS - 1)
    def _():
        pltpu.sync_copy(z_h.at[pl.ds(0, RPT)], acc.at[pl.ds(r0, RPT)])


def _write_own_rows(sid, cid, acc, out_h):
    r0 = pl.multiple_of(sid * RPT, 8)

    @pl.when(sid == NS - 1)
    def _():
        pltpu.sync_copy(acc.at[pl.ds(r0, RLAST)], out_h.at[cid, pl.ds(r0, RLAST)])

    @pl.when(sid < NS - 1)
    def _():
        pltpu.sync_copy(acc.at[pl.ds(r0, RPT)], out_h.at[cid, pl.ds(r0, RPT)])


def _sc_scatter(u, src, dst, zrows):
    width = u.shape[1]

    @functools.partial(
        pl.kernel,
        out_type=jax.ShapeDtypeStruct((NC, N, width), jnp.float32),
        mesh=_mesh(),
        scratch_types=[
            pltpu.VMEM((2, CH), jnp.int32),
            pltpu.VMEM((2, CH), jnp.int32),
            pltpu.VMEM((2, CH, width), jnp.float32),
            pltpu.VMEM_SHARED((N, width), jnp.float32),
            pltpu.SemaphoreType.DMA((2,)),
            pltpu.SemaphoreType.DMA((2,)),
        ])
    def k(u_h, src_h, dst_h, z_h, out_h, sidx, didx, rows, acc, gsem, isem):
        cid = lax.axis_index("c")
        sid = lax.axis_index("s")
        wid = cid * NS + sid
        _zero_own_rows(sid, z_h, acc)
        plsc.subcore_barrier()
        ebase = wid * EPW
        ncu = jnp.where(wid == NW - 1, NCH_LAST, NCH_A)

        pltpu.sync_copy(src_h.at[pl.ds(ebase, CH)], sidx.at[0])
        pltpu.sync_copy(dst_h.at[pl.ds(ebase, CH)], didx.at[0])
        b1_ = pl.multiple_of(ebase + CH, 8)
        pltpu.async_copy(src_h.at[pl.ds(b1_, CH)], sidx.at[1], isem.at[1])
        pltpu.async_copy(dst_h.at[pl.ds(b1_, CH)], didx.at[1], isem.at[1])
        pltpu.async_copy(u_h.at[sidx.at[0]], rows.at[0], gsem.at[0])

        @pl.loop(0, ncu)
        def _(j):
            p = lax.rem(j, 2)
            q = 1 - p

            @pl.when(j < ncu - 1)
            def _():
                pltpu.make_async_copy(
                    src_h.at[pl.ds(0, CH)], sidx.at[q], isem.at[q]).wait()
                pltpu.make_async_copy(
                    dst_h.at[pl.ds(0, CH)], didx.at[q], isem.at[q]).wait()
                pltpu.async_copy(u_h.at[sidx.at[q]], rows.at[q], gsem.at[q])

            pltpu.make_async_copy(
                u_h.at[pl.ds(0, CH)], rows.at[p], gsem.at[p]).wait()
            pltpu.sync_copy(rows.at[p], acc.at[didx.at[p]], add=True)

            @pl.when(j < ncu - 2)
            def _():
                nb = pl.multiple_of(ebase + (j + 2) * CH, 8)
                pltpu.async_copy(src_h.at[pl.ds(nb, CH)], sidx.at[p], isem.at[p])
                pltpu.async_copy(dst_h.at[pl.ds(nb, CH)], didx.at[p], isem.at[p])

        plsc.subcore_barrier()
        _write_own_rows(sid, cid, acc, out_h)

    return k(u, src, dst, zrows)


def _sc_scatter1(u, src, dst, zrows, ones_h):
    width = u.shape[1]

    @functools.partial(
        pl.kernel,
        out_type=(jax.ShapeDtypeStruct((NC, N, width), jnp.float32),
                  jax.ShapeDtypeStruct((NC, N, H), jnp.float32)),
        mesh=_mesh(),
        scratch_types=[
            pltpu.VMEM((2, CH), jnp.int32),
            pltpu.VMEM((2, CH), jnp.int32),
            pltpu.VMEM((2, CH, width), jnp.float32),
            pltpu.VMEM_SHARED((N, width), jnp.float32),
            pltpu.SemaphoreType.DMA((2,)),
            pltpu.SemaphoreType.DMA((2,)),
        ])
    def k(u_h, src_h, dst_h, z_h, ones_hbm, out_h, dout_h,
          sidx, didx, rows, acc, gsem, isem):
        cid = lax.axis_index("c")
        sid = lax.axis_index("s")
        wid = cid * NS + sid
        _zero_own_rows(sid, z_h, acc)
        pltpu.sync_copy(ones_hbm, rows.at[0])
        plsc.subcore_barrier()
        ebase = wid * EPW
        ncu = jnp.where(wid == NW - 1, NCH_LAST, NCH_A)

        pltpu.sync_copy(dst_h.at[pl.ds(ebase, CH)], didx.at[0])

        @pl.loop(0, ncu)
        def _(j):
            p = lax.rem(j, 2)
            q = 1 - p

            @pl.when(j < ncu - 1)
            def _():
                nb = pl.multiple_of(ebase + (j + 1) * CH, 8)
                pltpu.async_copy(dst_h.at[pl.ds(nb, CH)], didx.at[q], isem.at[q])

            pltpu.sync_copy(rows.at[0], acc.at[didx.at[p]], add=True)

            @pl.when(j < ncu - 1)
            def _():
                pltpu.make_async_copy(
                    dst_h.at[pl.ds(0, CH)], didx.at[q], isem.at[q]).wait()

        plsc.subcore_barrier()
        _write_own_rows(sid, cid, acc, dout_h)
        plsc.subcore_barrier()
        _zero_own_rows(sid, z_h, acc)
        plsc.subcore_barrier()

        pltpu.sync_copy(src_h.at[pl.ds(ebase, CH)], sidx.at[0])
        pltpu.sync_copy(dst_h.at[pl.ds(ebase, CH)], didx.at[0])
        b1_ = pl.multiple_of(ebase + CH, 8)
        pltpu.async_copy(src_h.at[pl.ds(b1_, CH)], sidx.at[1], isem.at[1])
        pltpu.async_copy(dst_h.at[pl.ds(b1_, CH)], didx.at[1], isem.at[1])
        pltpu.async_copy(u_h.at[sidx.at[0]], rows.at[0], gsem.at[0])

        @pl.loop(0, ncu)
        def _(j):
            p = lax.rem(j, 2)
            q = 1 - p

            @pl.when(j < ncu - 1)
            def _():
                pltpu.make_async_copy(
                    src_h.at[pl.ds(0, CH)], sidx.at[q], isem.at[q]).wait()
                pltpu.make_async_copy(
                    dst_h.at[pl.ds(0, CH)], didx.at[q], isem.at[q]).wait()
                pltpu.async_copy(u_h.at[sidx.at[q]], rows.at[q], gsem.at[q])

            pltpu.make_async_copy(
                u_h.at[pl.ds(0, CH)], rows.at[p], gsem.at[p]).wait()
            pltpu.sync_copy(rows.at[p], acc.at[didx.at[p]], add=True)

            @pl.when(j < ncu - 2)
            def _():
                nb = pl.multiple_of(ebase + (j + 2) * CH, 8)
                pltpu.async_copy(src_h.at[pl.ds(nb, CH)], sidx.at[p], isem.at[p])
                pltpu.async_copy(dst_h.at[pl.ds(nb, CH)], didx.at[p], isem.at[p])

        plsc.subcore_barrier()
        _write_own_rows(sid, cid, acc, out_h)

    return k(u, src, dst, zrows, ones_h)


def _sc_route_gather(ab, src, dst):
    @functools.partial(
        pl.kernel,
        out_type=jax.ShapeDtypeStruct((E, M), jnp.float32),
        mesh=_mesh(),
        scratch_types=[
            pltpu.VMEM((2, CH), jnp.int32),
            pltpu.VMEM((2, CH), jnp.int32),
            pltpu.VMEM((2, CH, H), jnp.float32),
            pltpu.VMEM((2, CH, H), jnp.float32),
            pltpu.VMEM((2, CH, M), jnp.float32),
            pltpu.SemaphoreType.DMA((2,)),
            pltpu.SemaphoreType.DMA((2,)),
            pltpu.SemaphoreType.DMA((2,)),
            pltpu.SemaphoreType.DMA((2,)),
        ])
    def k(ab_h, src_h, dst_h, g_h, sidx, didx, ra, rb, go, gsa, gsb, isem, wsem):
        cid = lax.axis_index("c")
        sid = lax.axis_index("s")
        wid = cid * NS + sid
        ebase = wid * EPW
        ncu = jnp.where(wid == NW - 1, NCH_LAST, NCH_A)

        pltpu.sync_copy(src_h.at[pl.ds(ebase, CH)], sidx.at[0])
        pltpu.sync_copy(dst_h.at[pl.ds(ebase, CH)], didx.at[0])
        b1_ = pl.multiple_of(ebase + CH, 8)
        pltpu.async_copy(src_h.at[pl.ds(b1_, CH)], sidx.at[1], isem.at[1])
        pltpu.async_copy(dst_h.at[pl.ds(b1_, CH)], didx.at[1], isem.at[1])
        pltpu.async_copy(ab_h.at[sidx.at[0]], ra.at[0], gsa.at[0])
        pltpu.async_copy(ab_h.at[didx.at[0]], rb.at[0], gsb.at[0])

        @pl.loop(0, ncu)
        def _(j):
            p = lax.rem(j, 2)
            q = 1 - p
            base = pl.multiple_of(ebase + j * CH, 8)

            @pl.when(j < ncu - 1)
            def _():
                pltpu.make_async_copy(
                    src_h.at[pl.ds(0, CH)], sidx.at[q], isem.at[q]).wait()
                pltpu.make_async_copy(
                    dst_h.at[pl.ds(0, CH)], didx.at[q], isem.at[q]).wait()
                pltpu.async_copy(ab_h.at[sidx.at[q]], ra.at[q], gsa.at[q])
                pltpu.async_copy(ab_h.at[didx.at[q]], rb.at[q], gsb.at[q])

            pltpu.make_async_copy(
                ab_h.at[pl.ds(0, CH)], ra.at[p], gsa.at[p]).wait()
            pltpu.make_async_copy(
                ab_h.at[pl.ds(0, CH)], rb.at[p], gsb.at[p]).wait()

            @pl.when(j >= 2)
            def _():
                pltpu.make_async_copy(
                    go.at[p], g_h.at[pl.ds(0, CH)], wsem.at[p]).wait()

            @pl.loop(0, CH, unroll=4)
            def _(r):
                for c in range(M // 16):
                    s = c * 16
                    go[p, r, pl.ds(s, 16)] = (
                        ra[p, r, pl.ds(s, 16)] + rb[p, r, pl.ds(M + s, 16)])

            pltpu.async_copy(go.at[p], g_h.at[pl.ds(base, CH)], wsem.at[p])

            @pl.when(j < ncu - 2)
            def _():
                nb = pl.multiple_of(ebase + (j + 2) * CH, 8)
                pltpu.async_copy(src_h.at[pl.ds(nb, CH)], sidx.at[p], isem.at[p])
                pltpu.async_copy(dst_h.at[pl.ds(nb, CH)], didx.at[p], isem.at[p])

        pltpu.make_async_copy(
            go.at[0], g_h.at[pl.ds(0, CH)], wsem.at[0]).wait()
        pltpu.make_async_copy(
            go.at[1], g_h.at[pl.ds(0, CH)], wsem.at[1]).wait()

    return k(ab, src, dst)


def _rep(shape):
    return pl.BlockSpec(shape, lambda i: tuple(0 for _ in shape))


def _tc_head(x, wlT, wdT, bias):
    def body(x_ref, wl_ref, wd_ref, b_ref, u_ref, d_ref):
        xb = x_ref[...]
        u_ref[...] = jnp.dot(xb, wl_ref[...], preferred_element_type=jnp.float32)
        d_ref[...] = jnp.dot(xb, wd_ref[...], preferred_element_type=jnp.float32) + b_ref[...]

    return pl.pallas_call(
        body,
        grid=(N // BN,),
        in_specs=[
            pl.BlockSpec((BN, IN), lambda i: (i, 0)),
            _rep((IN, H)), _rep((IN, H)), _rep((1, H)),
        ],
        out_specs=(pl.BlockSpec((BN, H), lambda i: (i, 0)),
                   pl.BlockSpec((BN, H), lambda i: (i, 0))),
        out_shape=(jax.ShapeDtypeStruct((N, H), jnp.float32),
                   jax.ShapeDtypeStruct((N, H), jnp.float32)),
    )(x, wlT, wdT, bias)


def _layer_out(sa, sb, dega, degb, d, g, be):
    deg = jnp.maximum(dega[:, :1] + degb[:, :1], 1.0)
    pre = (sa + sb) / deg + d
    mu = jnp.mean(pre, axis=-1, keepdims=True)
    var = jnp.mean((pre - mu) ** 2, axis=-1, keepdims=True)
    return jnp.maximum((pre - mu) * lax.rsqrt(var + 1e-5) * g + be, 0.0)


def _tc_mid(sa, sb, dega, degb, d, g, be, wlT, wdT, bias):
    win = d.shape[1]
    wu = wlT.shape[1]
    wd = wdT.shape[1]

    def body(sa_ref, sb_ref, da_ref, db_ref, d_ref, g_ref, be_ref,
             wl_ref, wd_ref, b_ref, u_ref, dn_ref):
        h = _layer_out(sa_ref[...], sb_ref[...], da_ref[...], db_ref[...],
                       d_ref[...], g_ref[...], be_ref[...])
        u_ref[...] = jnp.dot(h, wl_ref[...], preferred_element_type=jnp.float32)
        dn_ref[...] = jnp.dot(h, wd_ref[...], preferred_element_type=jnp.float32) + b_ref[...]

    return pl.pallas_call(
        body,
        grid=(N // BN,),
        in_specs=[
            pl.BlockSpec((BN, win), lambda i: (i, 0)),
            pl.BlockSpec((BN, win), lambda i: (i, 0)),
            pl.BlockSpec((BN, H), lambda i: (i, 0)),
            pl.BlockSpec((BN, H), lambda i: (i, 0)),
            pl.BlockSpec((BN, win), lambda i: (i, 0)),
            _rep((1, win)), _rep((1, win)),
            _rep((win, wu)), _rep((win, wd)), _rep((1, wd)),
        ],
        out_specs=(pl.BlockSpec((BN, wu), lambda i: (i, 0)),
                   pl.BlockSpec((BN, wd), lambda i: (i, 0))),
        out_shape=(jax.ShapeDtypeStruct((N, wu), jnp.float32),
                   jax.ShapeDtypeStruct((N, wd), jnp.float32)),
    )(sa, sb, dega, degb, d, g, be, wlT, wdT, bias)


def _tc_final(sa, sb, dega, degb, d3, hp, g3, be3,
              wihT, bih, whhT, bhh, vw1T, vb1, vw2, vb2,
              aw1T, ab1, aw2, ab2, rw1abT, rb1ab):
    grid = N // BN

    def body(sa_ref, sb_ref, da_ref, db_ref, d_ref, hp_ref, g_ref, be_ref,
             wih_ref, bih_ref, whh_ref, bhh_ref, vw1_ref, vb1_ref, vw2_ref, vb2_ref,
             aw1_ref, ab1_ref, aw2_ref, ab2_ref, rw1ab_ref, rb1ab_ref,
             ht_ref, vuln_ref, ab_ref, adapt_ref, acc_ref):
        i = pl.program_id(0)
        e = _layer_out(sa_ref[...][:, :M], sb_ref[...][:, :M],
                       da_ref[...], db_ref[...],
                       d_ref[...], g_ref[...], be_ref[...])
        hp_b = hp_ref[...]
        gi = jnp.dot(e, wih_ref[...], preferred_element_type=jnp.float32) + bih_ref[...]
        gh = jnp.dot(hp_b, whh_ref[...], preferred_element_type=jnp.float32) + bhh_ref[...]
        r = jax.nn.sigmoid(gi[:, :M] + gh[:, :M])
        z = jax.nn.sigmoid(gi[:, M:2 * M] + gh[:, M:2 * M])
        ng = jnp.tanh(gi[:, 2 * M:] + r * gh[:, 2 * M:])
        ht = (1.0 - z) * ng + z * hp_b
        ht_ref[...] = ht
        hv = jnp.maximum(jnp.dot(ht, vw1_ref[...], preferred_element_type=jnp.float32)
                         + vb1_ref[...], 0.0)
        vuln_ref[...] = jax.nn.sigmoid(
            jnp.sum(hv * vw2_ref[...], axis=-1, keepdims=True) + vb2_ref[...])
        ab_ref[...] = (jnp.dot(ht, rw1ab_ref[...], preferred_element_type=jnp.float32)
                       + rb1ab_ref[...])

        @pl.when(i == 0)
        def _():
            acc_ref[...] = jnp.zeros_like(acc_ref)

        acc_ref[...] += jnp.sum(ht, axis=0, keepdims=True)

        @pl.when(i == grid - 1)
        def _():
            pooled = acc_ref[...] / N
            t = jnp.maximum(jnp.dot(pooled, aw1_ref[...], preferred_element_type=jnp.float32)
                            + ab1_ref[...], 0.0)
            adapt_ref[...] = jax.nn.sigmoid(
                jnp.sum(t * aw2_ref[...], axis=-1, keepdims=True) + ab2_ref[...])

    return pl.pallas_call(
        body,
        grid=(grid,),
        in_specs=[
            pl.BlockSpec((BN, H), lambda i: (i, 0)),
            pl.BlockSpec((BN, H), lambda i: (i, 0)),
            pl.BlockSpec((BN, H), lambda i: (i, 0)),
            pl.BlockSpec((BN, H), lambda i: (i, 0)),
            pl.BlockSpec((BN, M), lambda i: (i, 0)),
            pl.BlockSpec((BN, M), lambda i: (i, 0)),
            _rep((1, M)), _rep((1, M)),
            _rep((M, 3 * M)), _rep((1, 3 * M)), _rep((M, 3 * M)), _rep((1, 3 * M)),
            _rep((M, M)), _rep((1, M)), _rep((1, M)), _rep((1, 1)),
            _rep((M, M)), _rep((1, M)), _rep((1, M)), _rep((1, 1)),
            _rep((M, H)), _rep((1, H)),
        ],
        out_specs=(pl.BlockSpec((BN, M), lambda i: (i, 0)),
                   pl.BlockSpec((BN, 1), lambda i: (i, 0)),
                   pl.BlockSpec((BN, H), lambda i: (i, 0)),
                   pl.BlockSpec((1, 1), lambda i: (0, 0))),
        out_shape=(jax.ShapeDtypeStruct((N, M), jnp.float32),
                   jax.ShapeDtypeStruct((N, 1), jnp.float32),
                   jax.ShapeDtypeStruct((N, H), jnp.float32),
                   jax.ShapeDtypeStruct((1, 1), jnp.float32)),
        scratch_shapes=[pltpu.VMEM((1, M), jnp.float32)],
    )(sa, sb, dega, degb, d3, hp, g3, be3, wihT, bih, whhT, bhh,
      vw1T, vb1, vw2, vb2, aw1T, ab1, aw2, ab2, rw1abT, rb1ab)


def _tc_route(g, ea, we, rw2, rb2):
    def body(g_ref, ea_ref, we_ref, rw2_ref, rb2_ref, out_ref):
        pre = jnp.maximum(g_ref[...] + ea_ref[...] * we_ref[...], 0.0)
        out_ref[...] = jax.nn.sigmoid(
            jnp.sum(pre * rw2_ref[...], axis=-1, keepdims=True) + rb2_ref[...])

    return pl.pallas_call(
        body,
        grid=(E // BE,),
        in_specs=[
            pl.BlockSpec((BE, M), lambda i: (i, 0)),
            pl.BlockSpec((BE, 1), lambda i: (i, 0)),
            _rep((1, M)), _rep((1, M)), _rep((1, 1)),
        ],
        out_specs=pl.BlockSpec((BE, 1), lambda i: (i, 0)),
        out_shape=jax.ShapeDtypeStruct((E, 1), jnp.float32),
    )(g, ea, we, rw2, rb2)


def kernel(x, edge_index, edge_attr, h_prev, params):
    p = params
    f32 = jnp.float32
    src = edge_index[0]
    dst = edge_index[1]

    z128 = jnp.zeros((RLAST, H), f32)
    ones128 = jnp.ones((CH, H), f32)

    w1lT = p['W1l'].T
    wd1T = (p['W1r'] + p['S1']).T
    b1 = p['b1l'][None, :]
    w2lT = p['W2l'].T
    wd2T = (p['W2r'] + p['S2']).T
    b2 = p['b2l'][None, :]
    w3lT_pad = jnp.concatenate(
        [p['W3l'].T, jnp.zeros((H, H - M), f32)], axis=1)
    wd3T = (p['W3r'] + p['S3']).T
    b3 = p['b3l'][None, :]
    g1 = p['g1'][None, :]; be1 = p['be1'][None, :]
    g2 = p['g2'][None, :]; be2 = p['be2'][None, :]
    g3 = p['g3'][None, :]; be3 = p['be3'][None, :]
    wihT = p['Wih'].T; bih = p['bih'][None, :]
    whhT = p['Whh'].T; bhh = p['bhh'][None, :]
    vw1T = p['Vw1'].T; vb1 = p['Vb1'][None, :]
    vw2 = p['Vw2']; vb2 = p['Vb2'][None, :]
    aw1T = p['Aw1'].T; ab1 = p['Ab1'][None, :]
    aw2 = p['Aw2']; ab2 = p['Ab2'][None, :]
    rw1abT = jnp.concatenate(
        [p['Rw1'][:, :M].T, p['Rw1'][:, M:2 * M].T], axis=1)
    rb1ab = jnp.concatenate([p['Rb1'], jnp.zeros((M,), f32)])[None, :]
    we = p['Rw1'][:, 2 * M:].T
    rw2 = p['Rw2']; rb2 = p['Rb2'][None, :]

    u1, d1 = _tc_head(x, w1lT, wd1T, b1)
    s1, degp = _sc_scatter1(u1, src, dst, z128, ones128)
    dega, degb = degp[0], degp[1]
    u2, d2 = _tc_mid(s1[0], s1[1], dega, degb, d1, g1, be1, w2lT, wd2T, b2)
    s2 = _sc_scatter(u2, src, dst, z128)
    u3, d3 = _tc_mid(s2[0], s2[1], dega, degb, d2, g2, be2, w3lT_pad, wd3T, b3)
    s3 = _sc_scatter(u3, src, dst, z128)
    ht, vuln, ab, adapt = _tc_final(
        s3[0], s3[1], dega, degb, d3, h_prev, g3, be3,
        wihT, bih, whhT, bhh, vw1T, vb1, vw2, vb2,
        aw1T, ab1, aw2, ab2, rw1abT, rb1ab)
    g = _sc_route_gather(ab, src, dst)
    route = _tc_route(g, edge_attr, we, rw2, rb2)
    return vuln[:, 0], adapt[:, 0], route[:, 0], ht


# speedup vs baseline: 1.0368x; 1.0017x over previous
import functools

import jax
import jax.numpy as jnp
from jax import lax
from jax.experimental import pallas as pl
from jax.experimental.pallas import tpu as pltpu
from jax.experimental.pallas import tpu_sc as plsc

N = 10000
E = 320000
IN = 128
H = 128
M = 64

NC = 2
NS = 16
NW = NC * NS
CH = 128
NCH_A = 78
NCH_LAST = 82
EPW = NCH_A * CH
RPT = 624
RLAST = N - (NS - 1) * RPT

BN = 2000
BE = 16000


def _mesh():
    return plsc.VectorSubcoreMesh(
        core_axis_name="c", subcore_axis_name="s",
        num_cores=NC, num_subcores=NS)


def _zero_own_rows(sid, z_h, acc):
    r0 = pl.multiple_of(sid * RPT, 8)

    @pl.when(sid == NS - 1)
    def _():
        pltpu.sync_copy(z_h, acc.at[pl.ds(r0, RLAST)])

    @pl.when(sid < NS - 1)
    def _():
        pltpu.sync_copy(z_h.at[pl.ds(0, RPT)], acc.at[pl.ds(r0, RPT)])


def _write_own_rows(sid, cid, acc, out_h):
    r0 = pl.multiple_of(sid * RPT, 8)

    @pl.when(sid == NS - 1)
    def _():
        pltpu.sync_copy(acc.at[pl.ds(r0, RLAST)], out_h.at[cid, pl.ds(r0, RLAST)])

    @pl.when(sid < NS - 1)
    def _():
        pltpu.sync_copy(acc.at[pl.ds(r0, RPT)], out_h.at[cid, pl.ds(r0, RPT)])


def _sc_scatter(u, src, dst, zrows):
    width = u.shape[1]

    @functools.partial(
        pl.kernel,
        out_type=jax.ShapeDtypeStruct((NC, N, width), jnp.float32),
        mesh=_mesh(),
        scratch_types=[
            pltpu.VMEM((2, CH), jnp.int32),
            pltpu.VMEM((2, CH), jnp.int32),
            pltpu.VMEM((2, CH, width), jnp.float32),
            pltpu.VMEM_SHARED((N, width), jnp.float32),
            pltpu.SemaphoreType.DMA((2,)),
            pltpu.SemaphoreType.DMA((2,)),
        ])
    def k(u_h, src_h, dst_h, z_h, out_h, sidx, didx, rows, acc, gsem, isem):
        cid = lax.axis_index("c")
        sid = lax.axis_index("s")
        wid = cid * NS + sid
        _zero_own_rows(sid, z_h, acc)
        plsc.subcore_barrier()
        ebase = wid * EPW
        ncu = jnp.where(wid == NW - 1, NCH_LAST, NCH_A)

        pltpu.sync_copy(src_h.at[pl.ds(ebase, CH)], sidx.at[0])
        pltpu.sync_copy(dst_h.at[pl.ds(ebase, CH)], didx.at[0])
        b1_ = pl.multiple_of(ebase + CH, 8)
        pltpu.async_copy(src_h.at[pl.ds(b1_, CH)], sidx.at[1], isem.at[1])
        pltpu.async_copy(dst_h.at[pl.ds(b1_, CH)], didx.at[1], isem.at[1])
        pltpu.async_copy(u_h.at[sidx.at[0]], rows.at[0], gsem.at[0])

        @pl.loop(0, ncu)
        def _(j):
            p = lax.rem(j, 2)
            q = 1 - p

            @pl.when(j < ncu - 1)
            def _():
                pltpu.make_async_copy(
                    src_h.at[pl.ds(0, CH)], sidx.at[q], isem.at[q]).wait()
                pltpu.make_async_copy(
                    dst_h.at[pl.ds(0, CH)], didx.at[q], isem.at[q]).wait()
                pltpu.async_copy(u_h.at[sidx.at[q]], rows.at[q], gsem.at[q])

            pltpu.make_async_copy(
                u_h.at[pl.ds(0, CH)], rows.at[p], gsem.at[p]).wait()
            pltpu.sync_copy(rows.at[p], acc.at[didx.at[p]], add=True)

            @pl.when(j < ncu - 2)
            def _():
                nb = pl.multiple_of(ebase + (j + 2) * CH, 8)
                pltpu.async_copy(src_h.at[pl.ds(nb, CH)], sidx.at[p], isem.at[p])
                pltpu.async_copy(dst_h.at[pl.ds(nb, CH)], didx.at[p], isem.at[p])

        plsc.subcore_barrier()
        _write_own_rows(sid, cid, acc, out_h)

    return k(u, src, dst, zrows)


def _sc_scatter1(u, src, dst, zrows, ones_h):
    width = u.shape[1]

    @functools.partial(
        pl.kernel,
        out_type=(jax.ShapeDtypeStruct((NC, N, width), jnp.float32),
                  jax.ShapeDtypeStruct((NC, N, H), jnp.float32)),
        mesh=_mesh(),
        scratch_types=[
            pltpu.VMEM((2, CH), jnp.int32),
            pltpu.VMEM((2, CH), jnp.int32),
            pltpu.VMEM((2, CH, width), jnp.float32),
            pltpu.VMEM_SHARED((N, width), jnp.float32),
            pltpu.SemaphoreType.DMA((2,)),
            pltpu.SemaphoreType.DMA((2,)),
        ])
    def k(u_h, src_h, dst_h, z_h, ones_hbm, out_h, dout_h,
          sidx, didx, rows, acc, gsem, isem):
        cid = lax.axis_index("c")
        sid = lax.axis_index("s")
        wid = cid * NS + sid
        _zero_own_rows(sid, z_h, acc)
        pltpu.sync_copy(ones_hbm, rows.at[0])
        plsc.subcore_barrier()
        ebase = wid * EPW
        ncu = jnp.where(wid == NW - 1, NCH_LAST, NCH_A)

        pltpu.sync_copy(dst_h.at[pl.ds(ebase, CH)], didx.at[0])

        @pl.loop(0, ncu)
        def _(j):
            p = lax.rem(j, 2)
            q = 1 - p

            @pl.when(j < ncu - 1)
            def _():
                nb = pl.multiple_of(ebase + (j + 1) * CH, 8)
                pltpu.async_copy(dst_h.at[pl.ds(nb, CH)], didx.at[q], isem.at[q])

            pltpu.sync_copy(rows.at[0], acc.at[didx.at[p]], add=True)

            @pl.when(j < ncu - 1)
            def _():
                pltpu.make_async_copy(
                    dst_h.at[pl.ds(0, CH)], didx.at[q], isem.at[q]).wait()

        plsc.subcore_barrier()
        _write_own_rows(sid, cid, acc, dout_h)
        plsc.subcore_barrier()
        _zero_own_rows(sid, z_h, acc)
        plsc.subcore_barrier()

        pltpu.sync_copy(src_h.at[pl.ds(ebase, CH)], sidx.at[0])
        pltpu.sync_copy(dst_h.at[pl.ds(ebase, CH)], didx.at[0])
        b1_ = pl.multiple_of(ebase + CH, 8)
        pltpu.async_copy(src_h.at[pl.ds(b1_, CH)], sidx.at[1], isem.at[1])
        pltpu.async_copy(dst_h.at[pl.ds(b1_, CH)], didx.at[1], isem.at[1])
        pltpu.async_copy(u_h.at[sidx.at[0]], rows.at[0], gsem.at[0])

        @pl.loop(0, ncu)
        def _(j):
            p = lax.rem(j, 2)
            q = 1 - p

            @pl.when(j < ncu - 1)
            def _():
                pltpu.make_async_copy(
                    src_h.at[pl.ds(0, CH)], sidx.at[q], isem.at[q]).wait()
                pltpu.make_async_copy(
                    dst_h.at[pl.ds(0, CH)], didx.at[q], isem.at[q]).wait()
                pltpu.async_copy(u_h.at[sidx.at[q]], rows.at[q], gsem.at[q])

            pltpu.make_async_copy(
                u_h.at[pl.ds(0, CH)], rows.at[p], gsem.at[p]).wait()
            pltpu.sync_copy(rows.at[p], acc.at[didx.at[p]], add=True)

            @pl.when(j < ncu - 2)
            def _():
                nb = pl.multiple_of(ebase + (j + 2) * CH, 8)
                pltpu.async_copy(src_h.at[pl.ds(nb, CH)], sidx.at[p], isem.at[p])
                pltpu.async_copy(dst_h.at[pl.ds(nb, CH)], didx.at[p], isem.at[p])

        plsc.subcore_barrier()
        _write_own_rows(sid, cid, acc, out_h)

    return k(u, src, dst, zrows, ones_h)


def _sc_route_gather(ab, src, dst):
    @functools.partial(
        pl.kernel,
        out_type=jax.ShapeDtypeStruct((E, M), jnp.float32),
        mesh=_mesh(),
        scratch_types=[
            pltpu.VMEM((2, CH), jnp.int32),
            pltpu.VMEM((2, CH), jnp.int32),
            pltpu.VMEM((2, CH, H), jnp.float32),
            pltpu.VMEM((2, CH, H), jnp.float32),
            pltpu.VMEM((2, CH, M), jnp.float32),
            pltpu.SemaphoreType.DMA((2,)),
            pltpu.SemaphoreType.DMA((2,)),
            pltpu.SemaphoreType.DMA((2,)),
            pltpu.SemaphoreType.DMA((2,)),
        ])
    def k(ab_h, src_h, dst_h, g_h, sidx, didx, ra, rb, go, gsa, gsb, isem, wsem):
        cid = lax.axis_index("c")
        sid = lax.axis_index("s")
        wid = cid * NS + sid
        ebase = wid * EPW
        ncu = jnp.where(wid == NW - 1, NCH_LAST, NCH_A)

        pltpu.sync_copy(src_h.at[pl.ds(ebase, CH)], sidx.at[0])
        pltpu.sync_copy(dst_h.at[pl.ds(ebase, CH)], didx.at[0])
        b1_ = pl.multiple_of(ebase + CH, 8)
        pltpu.async_copy(src_h.at[pl.ds(b1_, CH)], sidx.at[1], isem.at[1])
        pltpu.async_copy(dst_h.at[pl.ds(b1_, CH)], didx.at[1], isem.at[1])
        pltpu.async_copy(ab_h.at[sidx.at[0]], ra.at[0], gsa.at[0])
        pltpu.async_copy(ab_h.at[didx.at[0]], rb.at[0], gsb.at[0])

        @pl.loop(0, ncu)
        def _(j):
            p = lax.rem(j, 2)
            q = 1 - p
            base = pl.multiple_of(ebase + j * CH, 8)

            @pl.when(j < ncu - 1)
            def _():
                pltpu.make_async_copy(
                    src_h.at[pl.ds(0, CH)], sidx.at[q], isem.at[q]).wait()
                pltpu.make_async_copy(
                    dst_h.at[pl.ds(0, CH)], didx.at[q], isem.at[q]).wait()
                pltpu.async_copy(ab_h.at[sidx.at[q]], ra.at[q], gsa.at[q])
                pltpu.async_copy(ab_h.at[didx.at[q]], rb.at[q], gsb.at[q])

            pltpu.make_async_copy(
                ab_h.at[pl.ds(0, CH)], ra.at[p], gsa.at[p]).wait()
            pltpu.make_async_copy(
                ab_h.at[pl.ds(0, CH)], rb.at[p], gsb.at[p]).wait()

            @pl.when(j >= 2)
            def _():
                pltpu.make_async_copy(
                    go.at[p], g_h.at[pl.ds(0, CH)], wsem.at[p]).wait()

            @pl.loop(0, CH, unroll=4)
            def _(r):
                for c in range(M // 16):
                    s = c * 16
                    go[p, r, pl.ds(s, 16)] = (
                        ra[p, r, pl.ds(s, 16)] + rb[p, r, pl.ds(M + s, 16)])

            pltpu.async_copy(go.at[p], g_h.at[pl.ds(base, CH)], wsem.at[p])

            @pl.when(j < ncu - 2)
            def _():
                nb = pl.multiple_of(ebase + (j + 2) * CH, 8)
                pltpu.async_copy(src_h.at[pl.ds(nb, CH)], sidx.at[p], isem.at[p])
                pltpu.async_copy(dst_h.at[pl.ds(nb, CH)], didx.at[p], isem.at[p])

        pltpu.make_async_copy(
            go.at[0], g_h.at[pl.ds(0, CH)], wsem.at[0]).wait()
        pltpu.make_async_copy(
            go.at[1], g_h.at[pl.ds(0, CH)], wsem.at[1]).wait()

    return k(ab, src, dst)


def _rep(shape):
    return pl.BlockSpec(shape, lambda i: tuple(0 for _ in shape))


def _tc_head(x, wlT, wdT, bias):
    def body(x_ref, wl_ref, wd_ref, b_ref, u_ref, d_ref):
        xb = x_ref[...]
        u_ref[...] = jnp.dot(xb, wl_ref[...], preferred_element_type=jnp.float32)
        d_ref[...] = jnp.dot(xb, wd_ref[...], preferred_element_type=jnp.float32) + b_ref[...]

    return pl.pallas_call(
        body,
        grid=(N // BN,),
        in_specs=[
            pl.BlockSpec((BN, IN), lambda i: (i, 0)),
            _rep((IN, H)), _rep((IN, H)), _rep((1, H)),
        ],
        out_specs=(pl.BlockSpec((BN, H), lambda i: (i, 0)),
                   pl.BlockSpec((BN, H), lambda i: (i, 0))),
        out_shape=(jax.ShapeDtypeStruct((N, H), jnp.float32),
                   jax.ShapeDtypeStruct((N, H), jnp.float32)),
    )(x, wlT, wdT, bias)


def _layer_out(sa, sb, dega, degb, d, g, be):
    deg = jnp.maximum(dega[:, :1] + degb[:, :1], 1.0)
    pre = (sa + sb) / deg + d
    mu = jnp.mean(pre, axis=-1, keepdims=True)
    var = jnp.mean((pre - mu) ** 2, axis=-1, keepdims=True)
    return jnp.maximum((pre - mu) * lax.rsqrt(var + 1e-5) * g + be, 0.0)


def _tc_mid(sa, sb, dega, degb, d, g, be, wlT, wdT, bias):
    win = d.shape[1]
    wu = wlT.shape[1]
    wd = wdT.shape[1]

    def body(sa_ref, sb_ref, da_ref, db_ref, d_ref, g_ref, be_ref,
             wl_ref, wd_ref, b_ref, u_ref, dn_ref):
        h = _layer_out(sa_ref[...], sb_ref[...], da_ref[...], db_ref[...],
                       d_ref[...], g_ref[...], be_ref[...])
        u_ref[...] = jnp.dot(h, wl_ref[...], preferred_element_type=jnp.float32)
        dn_ref[...] = jnp.dot(h, wd_ref[...], preferred_element_type=jnp.float32) + b_ref[...]

    return pl.pallas_call(
        body,
        grid=(N // BN,),
        in_specs=[
            pl.BlockSpec((BN, win), lambda i: (i, 0)),
            pl.BlockSpec((BN, win), lambda i: (i, 0)),
            pl.BlockSpec((BN, H), lambda i: (i, 0)),
            pl.BlockSpec((BN, H), lambda i: (i, 0)),
            pl.BlockSpec((BN, win), lambda i: (i, 0)),
            _rep((1, win)), _rep((1, win)),
            _rep((win, wu)), _rep((win, wd)), _rep((1, wd)),
        ],
        out_specs=(pl.BlockSpec((BN, wu), lambda i: (i, 0)),
                   pl.BlockSpec((BN, wd), lambda i: (i, 0))),
        out_shape=(jax.ShapeDtypeStruct((N, wu), jnp.float32),
                   jax.ShapeDtypeStruct((N, wd), jnp.float32)),
    )(sa, sb, dega, degb, d, g, be, wlT, wdT, bias)


def _tc_final(sa, sb, dega, degb, d3, hp, g3, be3,
              wihT, bih, whhT, bhh, rw1abT, rb1ab):
    def body(sa_ref, sb_ref, da_ref, db_ref, d_ref, hp_ref, g_ref, be_ref,
             wih_ref, bih_ref, whh_ref, bhh_ref, rw1ab_ref, rb1ab_ref,
             ht_ref, ab_ref):
        e = _layer_out(sa_ref[...][:, :M], sb_ref[...][:, :M],
                       da_ref[...], db_ref[...],
                       d_ref[...], g_ref[...], be_ref[...])
        hp_b = hp_ref[...]
        gi = jnp.dot(e, wih_ref[...], preferred_element_type=jnp.float32) + bih_ref[...]
        gh = jnp.dot(hp_b, whh_ref[...], preferred_element_type=jnp.float32) + bhh_ref[...]
        r = jax.nn.sigmoid(gi[:, :M] + gh[:, :M])
        z = jax.nn.sigmoid(gi[:, M:2 * M] + gh[:, M:2 * M])
        ng = jnp.tanh(gi[:, 2 * M:] + r * gh[:, 2 * M:])
        ht = (1.0 - z) * ng + z * hp_b
        ht_ref[...] = ht
        ab_ref[...] = (jnp.dot(ht, rw1ab_ref[...], preferred_element_type=jnp.float32)
                       + rb1ab_ref[...])

    return pl.pallas_call(
        body,
        grid=(N // BN,),
        in_specs=[
            pl.BlockSpec((BN, H), lambda i: (i, 0)),
            pl.BlockSpec((BN, H), lambda i: (i, 0)),
            pl.BlockSpec((BN, H), lambda i: (i, 0)),
            pl.BlockSpec((BN, H), lambda i: (i, 0)),
            pl.BlockSpec((BN, M), lambda i: (i, 0)),
            pl.BlockSpec((BN, M), lambda i: (i, 0)),
            _rep((1, M)), _rep((1, M)),
            _rep((M, 3 * M)), _rep((1, 3 * M)), _rep((M, 3 * M)), _rep((1, 3 * M)),
            _rep((M, H)), _rep((1, H)),
        ],
        out_specs=(pl.BlockSpec((BN, M), lambda i: (i, 0)),
                   pl.BlockSpec((BN, H), lambda i: (i, 0))),
        out_shape=(jax.ShapeDtypeStruct((N, M), jnp.float32),
                   jax.ShapeDtypeStruct((N, H), jnp.float32)),
    )(sa, sb, dega, degb, d3, hp, g3, be3, wihT, bih, whhT, bhh, rw1abT, rb1ab)


def _tc_heads(ht, vw1T, vb1, vw2, vb2, aw1T, ab1, aw2, ab2):
    grid = N // BN

    def body(ht_ref, vw1_ref, vb1_ref, vw2_ref, vb2_ref,
             aw1_ref, ab1_ref, aw2_ref, ab2_ref,
             vuln_ref, adapt_ref, acc_ref):
        i = pl.program_id(0)
        ht = ht_ref[...]
        hv = jnp.maximum(jnp.dot(ht, vw1_ref[...], preferred_element_type=jnp.float32)
                         + vb1_ref[...], 0.0)
        vuln_ref[...] = jax.nn.sigmoid(
            jnp.sum(hv * vw2_ref[...], axis=-1, keepdims=True) + vb2_ref[...])

        @pl.when(i == 0)
        def _():
            acc_ref[...] = jnp.zeros_like(acc_ref)

        acc_ref[...] += jnp.sum(ht, axis=0, keepdims=True)

        @pl.when(i == grid - 1)
        def _():
            pooled = acc_ref[...] / N
            t = jnp.maximum(jnp.dot(pooled, aw1_ref[...], preferred_element_type=jnp.float32)
                            + ab1_ref[...], 0.0)
            adapt_ref[...] = jax.nn.sigmoid(
                jnp.sum(t * aw2_ref[...], axis=-1, keepdims=True) + ab2_ref[...])

    return pl.pallas_call(
        body,
        grid=(grid,),
        in_specs=[
            pl.BlockSpec((BN, M), lambda i: (i, 0)),
            _rep((M, M)), _rep((1, M)), _rep((1, M)), _rep((1, 1)),
            _rep((M, M)), _rep((1, M)), _rep((1, M)), _rep((1, 1)),
        ],
        out_specs=(pl.BlockSpec((BN, 1), lambda i: (i, 0)),
                   pl.BlockSpec((1, 1), lambda i: (0, 0))),
        out_shape=(jax.ShapeDtypeStruct((N, 1), jnp.float32),
                   jax.ShapeDtypeStruct((1, 1), jnp.float32)),
        scratch_shapes=[pltpu.VMEM((1, M), jnp.float32)],
    )(ht, vw1T, vb1, vw2, vb2, aw1T, ab1, aw2, ab2)


def _tc_route(g, ea, we, rw2, rb2):
    def body(g_ref, ea_ref, we_ref, rw2_ref, rb2_ref, out_ref):
        pre = jnp.maximum(g_ref[...] + ea_ref[...] * we_ref[...], 0.0)
        out_ref[...] = jax.nn.sigmoid(
            jnp.sum(pre * rw2_ref[...], axis=-1, keepdims=True) + rb2_ref[...])

    return pl.pallas_call(
        body,
        grid=(E // BE,),
        in_specs=[
            pl.BlockSpec((BE, M), lambda i: (i, 0)),
            pl.BlockSpec((BE, 1), lambda i: (i, 0)),
            _rep((1, M)), _rep((1, M)), _rep((1, 1)),
        ],
        out_specs=pl.BlockSpec((BE, 1), lambda i: (i, 0)),
        out_shape=jax.ShapeDtypeStruct((E, 1), jnp.float32),
    )(g, ea, we, rw2, rb2)


def kernel(x, edge_index, edge_attr, h_prev, params):
    p = params
    f32 = jnp.float32
    src = edge_index[0]
    dst = edge_index[1]

    z128 = jnp.zeros((RLAST, H), f32)
    ones128 = jnp.ones((CH, H), f32)

    w1lT = p['W1l'].T
    wd1T = (p['W1r'] + p['S1']).T
    b1 = p['b1l'][None, :]
    w2lT = p['W2l'].T
    wd2T = (p['W2r'] + p['S2']).T
    b2 = p['b2l'][None, :]
    w3lT_pad = jnp.concatenate(
        [p['W3l'].T, jnp.zeros((H, H - M), f32)], axis=1)
    wd3T = (p['W3r'] + p['S3']).T
    b3 = p['b3l'][None, :]
    g1 = p['g1'][None, :]; be1 = p['be1'][None, :]
    g2 = p['g2'][None, :]; be2 = p['be2'][None, :]
    g3 = p['g3'][None, :]; be3 = p['be3'][None, :]
    wihT = p['Wih'].T; bih = p['bih'][None, :]
    whhT = p['Whh'].T; bhh = p['bhh'][None, :]
    vw1T = p['Vw1'].T; vb1 = p['Vb1'][None, :]
    vw2 = p['Vw2']; vb2 = p['Vb2'][None, :]
    aw1T = p['Aw1'].T; ab1 = p['Ab1'][None, :]
    aw2 = p['Aw2']; ab2 = p['Ab2'][None, :]
    rw1abT = jnp.concatenate(
        [p['Rw1'][:, :M].T, p['Rw1'][:, M:2 * M].T], axis=1)
    rb1ab = jnp.concatenate([p['Rb1'], jnp.zeros((M,), f32)])[None, :]
    we = p['Rw1'][:, 2 * M:].T
    rw2 = p['Rw2']; rb2 = p['Rb2'][None, :]

    u1, d1 = _tc_head(x, w1lT, wd1T, b1)
    s1, degp = _sc_scatter1(u1, src, dst, z128, ones128)
    dega, degb = degp[0], degp[1]
    u2, d2 = _tc_mid(s1[0], s1[1], dega, degb, d1, g1, be1, w2lT, wd2T, b2)
    s2 = _sc_scatter(u2, src, dst, z128)
    u3, d3 = _tc_mid(s2[0], s2[1], dega, degb, d2, g2, be2, w3lT_pad, wd3T, b3)
    s3 = _sc_scatter(u3, src, dst, z128)
    ht, ab = _tc_final(
        s3[0], s3[1], dega, degb, d3, h_prev, g3, be3,
        wihT, bih, whhT, bhh, rw1abT, rb1ab)
    g = _sc_route_gather(ab, src, dst)
    vuln, adapt = _tc_heads(ht, vw1T, vb1, vw2, vb2, aw1T, ab1, aw2, ab2)
    route = _tc_route(g, edge_attr, we, rw2, rb2)
    return vuln[:, 0], adapt[:, 0], route[:, 0], ht
